# baseline (speedup 1.0000x reference)
import functools

import jax
import jax.numpy as jnp
import numpy as np
from jax.experimental import pallas as pl
from jax.experimental.pallas import tpu as pltpu
from jax.experimental.pallas import tpu_sc as plsc
from jax.sharding import Mesh, NamedSharding, PartitionSpec as P

DIM = 32
C = 8192
BN = 256
ROWS = 16 * 1024
GATHER_WINDOW = 256
PAD = 128


def _sim_body(x_ref, e_ref, dist_ref, ind_ref):
    x = x_ref[...]
    n = jnp.sqrt(jnp.sum(x * x, axis=-1, keepdims=True))
    xn = x / jnp.clip(n, 1e-12, None)
    e = e_ref[...]
    dist = jax.lax.dot_general(
        xn, e, (((1,), (1,)), ((), ())),
        preferred_element_type=jnp.float32,
    )
    ind_ref[0, 0, :] = jnp.argmax(dist, axis=-1).astype(jnp.int32)
    dist_ref[...] = dist


def _sc_gather(embed_pad, ind_flat, rows):
    mesh = plsc.VectorSubcoreMesh(core_axis_name="core",
                                  subcore_axis_name="subcore")

    @pl.kernel(out_type=jax.ShapeDtypeStruct((rows, PAD), jnp.float32),
               mesh=mesh)
    def gather_kernel(e_hbm, i_hbm, o_hbm):
        def body(i_vmem, o_vmem):
            pltpu.sync_copy(e_hbm.at[i_vmem.at[0]], o_vmem)

        pltpu.emit_pipeline(
            body,
            grid=(rows // GATHER_WINDOW,),
            in_specs=[pl.BlockSpec((1, GATHER_WINDOW),
                                   index_map=lambda i: (0, i))],
            out_specs=[pl.BlockSpec((GATHER_WINDOW, PAD),
                                    index_map=lambda i: (i, 0))],
            core_axis_name=("core", "subcore"),
            dimension_semantics=(pltpu.PARALLEL,),
        )(i_hbm, o_hbm)

    return gather_kernel(embed_pad, ind_flat)


def _shard_body(x_loc, embed):
    b, npts, d = x_loc.shape
    rows = b * npts
    nb = rows // BN
    xf = x_loc.reshape(rows, d)
    e2 = embed[0]
    dist, ind3 = pl.pallas_call(
        _sim_body,
        grid=(nb,),
        in_specs=[
            pl.BlockSpec((BN, DIM), lambda i: (i, 0)),
            pl.BlockSpec((C, DIM), lambda i: (0, 0)),
        ],
        out_specs=[
            pl.BlockSpec((BN, C), lambda i: (i, 0)),
            pl.BlockSpec((1, 1, BN), lambda i: (i, 0, 0)),
        ],
        out_shape=[
            jax.ShapeDtypeStruct((rows, C), jnp.float32),
            jax.ShapeDtypeStruct((nb, 1, BN), jnp.int32),
        ],
    )(xf, e2)
    e_pad = jnp.pad(e2, ((0, 0), (0, PAD - DIM)))
    q = _sc_gather(e_pad, ind3.reshape(1, rows), rows)[:, :DIM]
    return (q.reshape(b, npts, d),
            ind3.reshape(b, npts),
            dist.reshape(b, npts, C))


@jax.jit
def kernel(x, embed):
    devs = jax.devices()
    ndev = 2 if len(devs) >= 2 and x.shape[0] % 2 == 0 else 1
    mesh = Mesh(np.array(devs[:ndev]), ("dp",))
    xs = jax.lax.with_sharding_constraint(
        x, NamedSharding(mesh, P("dp", None, None)))
    es = jax.lax.with_sharding_constraint(
        embed, NamedSharding(mesh, P(None, None, None)))
    f = jax.shard_map(
        _shard_body, mesh=mesh,
        in_specs=(P("dp", None, None), P(None, None, None)),
        out_specs=(P("dp", None, None), P("dp", None), P("dp", None, None)),
        check_vma=False)
    return f(xs, es)


# speedup vs baseline: 1.6877x; 1.6877x over previous
import jax
import jax.numpy as jnp
from jax.experimental import pallas as pl
from jax.experimental.pallas import tpu as pltpu
from jax.experimental.pallas import tpu_sc as plsc

DIM = 32
C = 8192
BN = 256
ROWS = 16 * 1024
NB = ROWS // BN
GATHER_WINDOW = 256
PAD = 128


def _sim_body(x_ref, e_ref, dist_ref, ind_ref):
    x = x_ref[...]
    n = jnp.sqrt(jnp.sum(x * x, axis=-1, keepdims=True))
    xn = x / jnp.clip(n, 1e-12, None)
    e = e_ref[...]
    dist = jax.lax.dot_general(
        xn, e, (((1,), (1,)), ((), ())),
        preferred_element_type=jnp.float32,
    )
    ind_ref[0, 0, :] = jnp.argmax(dist, axis=-1).astype(jnp.int32)
    dist_ref[...] = dist


def _sc_gather(embed_pad, ind_flat):
    mesh = plsc.VectorSubcoreMesh(core_axis_name="core",
                                  subcore_axis_name="subcore")

    @pl.kernel(out_type=jax.ShapeDtypeStruct((ROWS, PAD), jnp.float32),
               mesh=mesh)
    def gather_kernel(e_hbm, i_hbm, o_hbm):
        def body(i_vmem, o_vmem):
            pltpu.sync_copy(e_hbm.at[i_vmem.at[0]], o_vmem)

        pltpu.emit_pipeline(
            body,
            grid=(ROWS // GATHER_WINDOW,),
            in_specs=[pl.BlockSpec((1, GATHER_WINDOW),
                                   index_map=lambda i: (0, i))],
            out_specs=[pl.BlockSpec((GATHER_WINDOW, PAD),
                                    index_map=lambda i: (i, 0))],
            core_axis_name=("core", "subcore"),
            dimension_semantics=(pltpu.PARALLEL,),
        )(i_hbm, o_hbm)

    return gather_kernel(embed_pad, ind_flat)


@jax.jit
def kernel(x, embed):
    b, npts, d = x.shape
    xf = x.reshape(b * npts, d)
    e2 = embed[0]
    dist, ind3 = pl.pallas_call(
        _sim_body,
        grid=(NB,),
        in_specs=[
            pl.BlockSpec((BN, DIM), lambda i: (i, 0)),
            pl.BlockSpec((C, DIM), lambda i: (0, 0)),
        ],
        out_specs=[
            pl.BlockSpec((BN, C), lambda i: (i, 0)),
            pl.BlockSpec((1, 1, BN), lambda i: (i, 0, 0)),
        ],
        out_shape=[
            jax.ShapeDtypeStruct((ROWS, C), jnp.float32),
            jax.ShapeDtypeStruct((NB, 1, BN), jnp.int32),
        ],
    )(xf, e2)
    e_pad = jnp.pad(e2, ((0, 0), (0, PAD - DIM)))
    q = _sc_gather(e_pad, ind3.reshape(1, ROWS))[:, :DIM]
    return (q.reshape(b, npts, d),
            ind3.reshape(b, npts),
            dist.reshape(b, npts, C))
